# Initial kernel scaffold; baseline (speedup 1.0000x reference)
import functools

import jax
import jax.numpy as jnp
from jax import lax
from jax.experimental import pallas as pl
from jax.experimental.pallas import tpu as pltpu
from jax.experimental.pallas import tpu_sc as plsc

N_NODES = 10000
NP = 10240
E = 320000
EP = 327680
D = 64
NUM_GRAPHS = 128
NC, NS = 2, 16
ROWS_PER_TILE = NP // NS
CHUNKS = EP // (NC * NS) // 128


def _zero_fill(buf, n_rows, width):
    zero16 = jnp.zeros((16,), jnp.float32)
    cols = width // 16

    def body(i, _):
        r = i // cols
        c = (i % cols) * 16
        buf[r, pl.ds(c, 16)] = zero16
        return 0

    lax.fori_loop(0, n_rows * cols, body, 0)


def _scatter_kernel(g_hbm, src_hbm, dst_hbm, out_hbm, src_v, dst_v, rows_v,
                    zbuf_v, acc_sh, sem):
    c = lax.axis_index("c")
    s = lax.axis_index("s")
    wid = c * NS + s

    _zero_fill(zbuf_v, 64, D)
    base = s * ROWS_PER_TILE
    for k in range(ROWS_PER_TILE // 64):
        pltpu.sync_copy(zbuf_v, acc_sh.at[pl.ds(base + k * 64, 64)])

    pltpu.sync_copy(src_hbm.at[wid], src_v)
    pltpu.sync_copy(dst_hbm.at[wid], dst_v)
    plsc.subcore_barrier()

    def body(j, _):
        pltpu.async_copy(g_hbm.at[src_v.at[j]], rows_v, sem).wait()
        pltpu.sync_copy(rows_v, acc_sh.at[dst_v.at[j]], add=True)
        return 0

    lax.fori_loop(0, CHUNKS, body, 0)
    plsc.subcore_barrier()

    pltpu.sync_copy(acc_sh.at[pl.ds(base, ROWS_PER_TILE)],
                    out_hbm.at[c, pl.ds(base, ROWS_PER_TILE)])


def _sc_scatter(g, src3, dst3):
    mesh = plsc.VectorSubcoreMesh(core_axis_name="c", subcore_axis_name="s")
    return pl.kernel(
        _scatter_kernel,
        mesh=mesh,
        out_type=jax.ShapeDtypeStruct((NC, NP, D), jnp.float32),
        scratch_types=[
            pltpu.VMEM((CHUNKS, 128), jnp.int32),
            pltpu.VMEM((CHUNKS, 128), jnp.int32),
            pltpu.VMEM((128, D), jnp.float32),
            pltpu.VMEM((64, D), jnp.float32),
            pltpu.VMEM_SHARED((NP, D), jnp.float32),
            pltpu.SemaphoreType.DMA,
        ],
    )(g, src3, dst3)


def _deg_kernel(dst_hbm, out_hbm, dst_v, ones_v, zbuf_v, acc_sh):
    c = lax.axis_index("c")
    s = lax.axis_index("s")
    wid = c * NS + s

    _zero_fill(zbuf_v, 64, 16)
    one16 = jnp.ones((16,), jnp.float32)

    def fill_ones(i, _):
        ones_v[i, :] = one16
        return 0

    lax.fori_loop(0, 128, fill_ones, 0)

    base = s * ROWS_PER_TILE
    for k in range(ROWS_PER_TILE // 64):
        pltpu.sync_copy(zbuf_v, acc_sh.at[pl.ds(base + k * 64, 64)])
    pltpu.sync_copy(dst_hbm.at[wid], dst_v)
    plsc.subcore_barrier()

    def body(j, _):
        pltpu.sync_copy(ones_v, acc_sh.at[dst_v.at[j]], add=True)
        return 0

    lax.fori_loop(0, CHUNKS, body, 0)
    plsc.subcore_barrier()
    pltpu.sync_copy(acc_sh.at[pl.ds(base, ROWS_PER_TILE)],
                    out_hbm.at[c, pl.ds(base, ROWS_PER_TILE)])


def _sc_deg(dst3):
    mesh = plsc.VectorSubcoreMesh(core_axis_name="c", subcore_axis_name="s")
    return pl.kernel(
        _deg_kernel,
        mesh=mesh,
        out_type=jax.ShapeDtypeStruct((NC, NP, 16), jnp.float32),
        scratch_types=[
            pltpu.VMEM((CHUNKS, 128), jnp.int32),
            pltpu.VMEM((128, 16), jnp.float32),
            pltpu.VMEM((64, 16), jnp.float32),
            pltpu.VMEM_SHARED((NP, 16), jnp.float32),
        ],
    )(dst3)


def _dinv_body(degp_ref, o_ref):
    deg = degp_ref[0, :, 0:1] + degp_ref[1, :, 0:1] + 1.0
    o_ref[...] = lax.rsqrt(deg)


def _tc_dinv(deg_partials):
    return pl.pallas_call(
        _dinv_body,
        out_shape=jax.ShapeDtypeStruct((NP, 1), jnp.float32),
    )(deg_partials)


def _g0_body(x_ref, w_ref, dinv_ref, o_ref):
    o_ref[...] = dinv_ref[...] * jnp.dot(
        x_ref[...], w_ref[...], preferred_element_type=jnp.float32)


def _tc_g0(x, W0, dinv):
    blk = 512
    grid = NP // blk
    return pl.pallas_call(
        _g0_body,
        grid=(grid,),
        in_specs=[
            pl.BlockSpec((blk, x.shape[1]), lambda i: (i, 0)),
            pl.BlockSpec((x.shape[1], D), lambda i: (0, 0)),
            pl.BlockSpec((blk, 1), lambda i: (i, 0)),
        ],
        out_specs=pl.BlockSpec((blk, D), lambda i: (i, 0)),
        out_shape=jax.ShapeDtypeStruct((NP, D), jnp.float32),
    )(x, W0, dinv)


def _mid_body(s_ref, g_ref, dinv_ref, b_ref, w_ref, o_ref):
    dinv = dinv_ref[...]
    a = dinv * (s_ref[0] + s_ref[1] + g_ref[...]) + b_ref[...]
    r = jnp.maximum(a, 0.0)
    o_ref[...] = dinv * jnp.dot(r, w_ref[...],
                                preferred_element_type=jnp.float32)


def _tc_mid(S, g, dinv, b, W):
    blk = 512
    grid = NP // blk
    return pl.pallas_call(
        _mid_body,
        grid=(grid,),
        in_specs=[
            pl.BlockSpec((NC, blk, D), lambda i: (0, i, 0)),
            pl.BlockSpec((blk, D), lambda i: (i, 0)),
            pl.BlockSpec((blk, 1), lambda i: (i, 0)),
            pl.BlockSpec((1, D), lambda i: (0, 0)),
            pl.BlockSpec((D, D), lambda i: (0, 0)),
        ],
        out_specs=pl.BlockSpec((blk, D), lambda i: (i, 0)),
        out_shape=jax.ShapeDtypeStruct((NP, D), jnp.float32),
    )(S, g, dinv, b, W)


def _final_body(s_ref, g_ref, dinv_ref, b_ref, batch_ref, wlin_ref, blin_ref,
                o_ref, sums_ref, cnts_ref):
    i = pl.program_id(0)

    @pl.when(i == 0)
    def _():
        sums_ref[...] = jnp.zeros_like(sums_ref)
        cnts_ref[...] = jnp.zeros_like(cnts_ref)

    dinv = dinv_ref[...]
    a = dinv * (s_ref[0] + s_ref[1] + g_ref[...]) + b_ref[...]
    r = jnp.maximum(a, 0.0)
    bt = batch_ref[...].reshape(1, -1)
    gid = lax.broadcasted_iota(jnp.int32, (NUM_GRAPHS, bt.shape[1]), 0)
    oh = (gid == bt).astype(jnp.float32)
    sums_ref[...] += jnp.dot(oh, r, preferred_element_type=jnp.float32)
    cnts_ref[...] += jnp.sum(oh, axis=1, keepdims=True)

    @pl.when(i == pl.num_programs(0) - 1)
    def _():
        pooled = sums_ref[...] / jnp.maximum(cnts_ref[...], 1.0)
        o_ref[...] = jnp.dot(pooled, wlin_ref[...],
                             preferred_element_type=jnp.float32) + blin_ref[...]


def _tc_final(S, g, dinv, b, batch2, Wlin, blin):
    blk = 512
    grid = NP // blk
    return pl.pallas_call(
        _final_body,
        grid=(grid,),
        in_specs=[
            pl.BlockSpec((NC, blk, D), lambda i: (0, i, 0)),
            pl.BlockSpec((blk, D), lambda i: (i, 0)),
            pl.BlockSpec((blk, 1), lambda i: (i, 0)),
            pl.BlockSpec((1, D), lambda i: (0, 0)),
            pl.BlockSpec((blk, 1), lambda i: (i, 0)),
            pl.BlockSpec((D, Wlin.shape[1]), lambda i: (0, 0)),
            pl.BlockSpec((1, Wlin.shape[1]), lambda i: (0, 0)),
        ],
        out_specs=pl.BlockSpec((NUM_GRAPHS, Wlin.shape[1]), lambda i: (0, 0)),
        out_shape=jax.ShapeDtypeStruct((NUM_GRAPHS, Wlin.shape[1]), jnp.float32),
        scratch_shapes=[
            pltpu.VMEM((NUM_GRAPHS, D), jnp.float32),
            pltpu.VMEM((NUM_GRAPHS, 1), jnp.float32),
        ],
    )(S, g, dinv, b, batch2, Wlin, blin)


@jax.jit
def kernel(x, edge_index, batch, W0, b0, W1, b1, W2, b2, Wlin, blin):
    n = x.shape[0]
    xp = jnp.zeros((NP, x.shape[1]), x.dtype).at[:n].set(x)
    src = jnp.full((EP,), n, jnp.int32).at[:E].set(edge_index[0])
    dst = jnp.full((EP,), NP - 1, jnp.int32).at[:E].set(edge_index[1])
    src3 = src.reshape(NC * NS, CHUNKS, 128)
    dst3 = dst.reshape(NC * NS, CHUNKS, 128)
    batch2 = jnp.full((NP, 1), NUM_GRAPHS + 7, jnp.int32).at[:n, 0].set(batch)

    deg_partials = _sc_deg(dst3)
    dinv = _tc_dinv(deg_partials)

    g0 = _tc_g0(xp, W0, dinv)
    S0 = _sc_scatter(g0, src3, dst3)
    g1 = _tc_mid(S0, g0, dinv, b0.reshape(1, D), W1)
    S1 = _sc_scatter(g1, src3, dst3)
    g2 = _tc_mid(S1, g1, dinv, b1.reshape(1, D), W2)
    S2 = _sc_scatter(g2, src3, dst3)
    out = _tc_final(S2, g2, dinv, b2.reshape(1, D), batch2,
                    Wlin, blin.reshape(1, -1))
    return out


# speedup vs baseline: 12.6684x; 12.6684x over previous
import functools

import jax
import jax.numpy as jnp
from jax import lax
from jax.experimental import pallas as pl
from jax.experimental.pallas import tpu as pltpu
from jax.experimental.pallas import tpu_sc as plsc

N_NODES = 10000
NP = 10240
E = 320000
EP = 327680
D = 64
NUM_GRAPHS = 128
NC, NS = 2, 16
ROWS_PER_TILE = NP // NS
CHUNKS = EP // (NC * NS) // 128


def _zero_fill(buf, n_rows, width):
    zero16 = jnp.zeros((16,), jnp.float32)
    cols = width // 16

    def body(i, _):
        r = i // cols
        c = (i % cols) * 16
        buf[r, pl.ds(c, 16)] = zero16
        return 0

    lax.fori_loop(0, n_rows * cols, body, 0)


def _scatter_kernel(g_hbm, src_hbm, dst_hbm, out_hbm, src_v, dst_v, rows_v,
                    zbuf_v, acc_sh, sem):
    c = lax.axis_index("c")
    s = lax.axis_index("s")
    wid = c * NS + s

    _zero_fill(zbuf_v, 64, D)
    base = s * ROWS_PER_TILE
    for k in range(ROWS_PER_TILE // 64):
        pltpu.sync_copy(zbuf_v, acc_sh.at[pl.ds(base + k * 64, 64)])

    pltpu.sync_copy(src_hbm.at[wid], src_v)
    pltpu.sync_copy(dst_hbm.at[wid], dst_v)
    plsc.subcore_barrier()

    def body(j, _):
        pltpu.async_copy(g_hbm.at[src_v.at[j]], rows_v, sem).wait()
        pltpu.sync_copy(rows_v, acc_sh.at[dst_v.at[j]], add=True)
        return 0

    lax.fori_loop(0, CHUNKS, body, 0)
    plsc.subcore_barrier()

    pltpu.sync_copy(acc_sh.at[pl.ds(base, ROWS_PER_TILE)],
                    out_hbm.at[c, pl.ds(base, ROWS_PER_TILE)])


def _sc_scatter(g, src3, dst3):
    mesh = plsc.VectorSubcoreMesh(core_axis_name="c", subcore_axis_name="s")
    return pl.kernel(
        _scatter_kernel,
        mesh=mesh,
        compiler_params=pltpu.CompilerParams(use_tc_tiling_on_sc=False),
        out_type=jax.ShapeDtypeStruct((NC, NP, D), jnp.float32),
        scratch_types=[
            pltpu.VMEM((CHUNKS, 128), jnp.int32),
            pltpu.VMEM((CHUNKS, 128), jnp.int32),
            pltpu.VMEM((128, D), jnp.float32),
            pltpu.VMEM((64, D), jnp.float32),
            pltpu.VMEM_SHARED((NP, D), jnp.float32),
            pltpu.SemaphoreType.DMA,
        ],
    )(g, src3, dst3)


def _deg_kernel(dst_hbm, out_hbm, dst_v, ones_v, zbuf_v, acc_sh):
    c = lax.axis_index("c")
    s = lax.axis_index("s")
    wid = c * NS + s

    _zero_fill(zbuf_v, 64, 16)
    one16 = jnp.ones((16,), jnp.float32)

    def fill_ones(i, _):
        ones_v[i, :] = one16
        return 0

    lax.fori_loop(0, 128, fill_ones, 0)

    base = s * ROWS_PER_TILE
    for k in range(ROWS_PER_TILE // 64):
        pltpu.sync_copy(zbuf_v, acc_sh.at[pl.ds(base + k * 64, 64)])
    pltpu.sync_copy(dst_hbm.at[wid], dst_v)
    plsc.subcore_barrier()

    def body(j, _):
        pltpu.sync_copy(ones_v, acc_sh.at[dst_v.at[j]], add=True)
        return 0

    lax.fori_loop(0, CHUNKS, body, 0)
    plsc.subcore_barrier()
    pltpu.sync_copy(acc_sh.at[pl.ds(base, ROWS_PER_TILE)],
                    out_hbm.at[c, pl.ds(base, ROWS_PER_TILE)])


def _sc_deg(dst3):
    mesh = plsc.VectorSubcoreMesh(core_axis_name="c", subcore_axis_name="s")
    return pl.kernel(
        _deg_kernel,
        mesh=mesh,
        compiler_params=pltpu.CompilerParams(use_tc_tiling_on_sc=False),
        out_type=jax.ShapeDtypeStruct((NC, NP, 16), jnp.float32),
        scratch_types=[
            pltpu.VMEM((CHUNKS, 128), jnp.int32),
            pltpu.VMEM((128, 16), jnp.float32),
            pltpu.VMEM((64, 16), jnp.float32),
            pltpu.VMEM_SHARED((NP, 16), jnp.float32),
        ],
    )(dst3)


def _dinv_body(degp_ref, o_ref):
    deg = degp_ref[0, :, 0:1] + degp_ref[1, :, 0:1] + 1.0
    o_ref[...] = lax.rsqrt(deg)


def _tc_dinv(deg_partials):
    return pl.pallas_call(
        _dinv_body,
        out_shape=jax.ShapeDtypeStruct((NP, 1), jnp.float32),
    )(deg_partials)


def _g0_body(x_ref, w_ref, dinv_ref, o_ref):
    o_ref[...] = dinv_ref[...] * jnp.dot(
        x_ref[...], w_ref[...], preferred_element_type=jnp.float32)


def _tc_g0(x, W0, dinv):
    blk = 512
    grid = NP // blk
    return pl.pallas_call(
        _g0_body,
        grid=(grid,),
        in_specs=[
            pl.BlockSpec((blk, x.shape[1]), lambda i: (i, 0)),
            pl.BlockSpec((x.shape[1], D), lambda i: (0, 0)),
            pl.BlockSpec((blk, 1), lambda i: (i, 0)),
        ],
        out_specs=pl.BlockSpec((blk, D), lambda i: (i, 0)),
        out_shape=jax.ShapeDtypeStruct((NP, D), jnp.float32),
    )(x, W0, dinv)


def _mid_body(s_ref, g_ref, dinv_ref, b_ref, w_ref, o_ref):
    dinv = dinv_ref[...]
    a = dinv * (s_ref[0] + s_ref[1] + g_ref[...]) + b_ref[...]
    r = jnp.maximum(a, 0.0)
    o_ref[...] = dinv * jnp.dot(r, w_ref[...],
                                preferred_element_type=jnp.float32)


def _tc_mid(S, g, dinv, b, W):
    blk = 512
    grid = NP // blk
    return pl.pallas_call(
        _mid_body,
        grid=(grid,),
        in_specs=[
            pl.BlockSpec((NC, blk, D), lambda i: (0, i, 0)),
            pl.BlockSpec((blk, D), lambda i: (i, 0)),
            pl.BlockSpec((blk, 1), lambda i: (i, 0)),
            pl.BlockSpec((1, D), lambda i: (0, 0)),
            pl.BlockSpec((D, D), lambda i: (0, 0)),
        ],
        out_specs=pl.BlockSpec((blk, D), lambda i: (i, 0)),
        out_shape=jax.ShapeDtypeStruct((NP, D), jnp.float32),
    )(S, g, dinv, b, W)


def _final_body(s_ref, g_ref, dinv_ref, b_ref, batch_ref, wlin_ref, blin_ref,
                o_ref, sums_ref, cnts_ref):
    i = pl.program_id(0)

    @pl.when(i == 0)
    def _():
        sums_ref[...] = jnp.zeros_like(sums_ref)
        cnts_ref[...] = jnp.zeros_like(cnts_ref)

    dinv = dinv_ref[...]
    a = dinv * (s_ref[0] + s_ref[1] + g_ref[...]) + b_ref[...]
    r = jnp.maximum(a, 0.0)
    bt = batch_ref[...].reshape(1, -1)
    gid = lax.broadcasted_iota(jnp.int32, (NUM_GRAPHS, bt.shape[1]), 0)
    oh = (gid == bt).astype(jnp.float32)
    sums_ref[...] += jnp.dot(oh, r, preferred_element_type=jnp.float32)
    cnts_ref[...] += jnp.sum(oh, axis=1, keepdims=True)

    @pl.when(i == pl.num_programs(0) - 1)
    def _():
        pooled = sums_ref[...] / jnp.maximum(cnts_ref[...], 1.0)
        o_ref[...] = jnp.dot(pooled, wlin_ref[...],
                             preferred_element_type=jnp.float32) + blin_ref[...]


def _tc_final(S, g, dinv, b, batch2, Wlin, blin):
    blk = 512
    grid = NP // blk
    return pl.pallas_call(
        _final_body,
        grid=(grid,),
        in_specs=[
            pl.BlockSpec((NC, blk, D), lambda i: (0, i, 0)),
            pl.BlockSpec((blk, D), lambda i: (i, 0)),
            pl.BlockSpec((blk, 1), lambda i: (i, 0)),
            pl.BlockSpec((1, D), lambda i: (0, 0)),
            pl.BlockSpec((blk, 1), lambda i: (i, 0)),
            pl.BlockSpec((D, Wlin.shape[1]), lambda i: (0, 0)),
            pl.BlockSpec((1, Wlin.shape[1]), lambda i: (0, 0)),
        ],
        out_specs=pl.BlockSpec((NUM_GRAPHS, Wlin.shape[1]), lambda i: (0, 0)),
        out_shape=jax.ShapeDtypeStruct((NUM_GRAPHS, Wlin.shape[1]), jnp.float32),
        scratch_shapes=[
            pltpu.VMEM((NUM_GRAPHS, D), jnp.float32),
            pltpu.VMEM((NUM_GRAPHS, 1), jnp.float32),
        ],
    )(S, g, dinv, b, batch2, Wlin, blin)


@jax.jit
def kernel(x, edge_index, batch, W0, b0, W1, b1, W2, b2, Wlin, blin):
    n = x.shape[0]
    xp = jnp.zeros((NP, x.shape[1]), x.dtype).at[:n].set(x)
    src = jnp.full((EP,), n, jnp.int32).at[:E].set(edge_index[0])
    dst = jnp.full((EP,), NP - 1, jnp.int32).at[:E].set(edge_index[1])
    src3 = src.reshape(NC * NS, CHUNKS, 128)
    dst3 = dst.reshape(NC * NS, CHUNKS, 128)
    batch2 = jnp.full((NP, 1), NUM_GRAPHS + 7, jnp.int32).at[:n, 0].set(batch)

    deg_partials = _sc_deg(dst3)
    dinv = _tc_dinv(deg_partials)

    g0 = _tc_g0(xp, W0, dinv)
    S0 = _sc_scatter(g0, src3, dst3)
    g1 = _tc_mid(S0, g0, dinv, b0.reshape(1, D), W1)
    S1 = _sc_scatter(g1, src3, dst3)
    g2 = _tc_mid(S1, g1, dinv, b1.reshape(1, D), W2)
    S2 = _sc_scatter(g2, src3, dst3)
    out = _tc_final(S2, g2, dinv, b2.reshape(1, D), batch2,
                    Wlin, blin.reshape(1, -1))
    return out


# speedup vs baseline: 14.9888x; 1.1832x over previous
import functools

import jax
import jax.numpy as jnp
from jax import lax
from jax.experimental import pallas as pl
from jax.experimental.pallas import tpu as pltpu
from jax.experimental.pallas import tpu_sc as plsc

N_NODES = 10000
NP = 10240
E = 320000
EP = 327680
D = 64
NUM_GRAPHS = 128
NC, NS = 2, 16
ROWS_PER_TILE = NP // NS
CHUNKS = EP // (NC * NS) // 128
NBUF = 4


def _zero_fill(buf, n_rows, width):
    zero16 = jnp.zeros((16,), jnp.float32)
    cols = width // 16

    def body(i, _):
        r = i // cols
        c = (i % cols) * 16
        buf[r, pl.ds(c, 16)] = zero16
        return 0

    lax.fori_loop(0, n_rows * cols, body, 0)


def _scatter_kernel(g_hbm, src_hbm, dst_hbm, out_hbm, src_v, dst_v, rows_v,
                    zbuf_v, acc_sh, sem):
    c = lax.axis_index("c")
    s = lax.axis_index("s")
    wid = c * NS + s

    _zero_fill(zbuf_v, 64, D)
    base = s * ROWS_PER_TILE
    for k in range(ROWS_PER_TILE // 64):
        pltpu.sync_copy(zbuf_v, acc_sh.at[pl.ds(base + k * 64, 64)])

    pltpu.sync_copy(src_hbm.at[wid], src_v)
    pltpu.sync_copy(dst_hbm.at[wid], dst_v)
    plsc.subcore_barrier()

    for b in range(NBUF):
        pltpu.async_copy(g_hbm.at[src_v.at[b]], rows_v.at[b], sem[b])

    def body(g, _):
        for b in range(NBUF):
            j = g * NBUF + b
            pltpu.make_async_copy(g_hbm.at[src_v.at[j]], rows_v.at[b],
                                  sem[b]).wait()
            pltpu.sync_copy(rows_v.at[b], acc_sh.at[dst_v.at[j]], add=True)

            @pl.when(j + NBUF < CHUNKS)
            def _():
                pltpu.async_copy(g_hbm.at[src_v.at[j + NBUF]], rows_v.at[b],
                                 sem[b])
        return 0

    lax.fori_loop(0, CHUNKS // NBUF, body, 0)
    plsc.subcore_barrier()

    pltpu.sync_copy(acc_sh.at[pl.ds(base, ROWS_PER_TILE)],
                    out_hbm.at[c, pl.ds(base, ROWS_PER_TILE)])


def _sc_scatter(g, src3, dst3):
    mesh = plsc.VectorSubcoreMesh(core_axis_name="c", subcore_axis_name="s")
    return pl.kernel(
        _scatter_kernel,
        mesh=mesh,
        compiler_params=pltpu.CompilerParams(use_tc_tiling_on_sc=False),
        out_type=jax.ShapeDtypeStruct((NC, NP, D), jnp.float32),
        scratch_types=[
            pltpu.VMEM((CHUNKS, 128), jnp.int32),
            pltpu.VMEM((CHUNKS, 128), jnp.int32),
            pltpu.VMEM((NBUF, 128, D), jnp.float32),
            pltpu.VMEM((64, D), jnp.float32),
            pltpu.VMEM_SHARED((NP, D), jnp.float32),
            [pltpu.SemaphoreType.DMA] * NBUF,
        ],
    )(g, src3, dst3)


def _deg_kernel(dst_hbm, out_hbm, dst_v, ones_v, zbuf_v, acc_sh):
    c = lax.axis_index("c")
    s = lax.axis_index("s")
    wid = c * NS + s

    _zero_fill(zbuf_v, 64, 16)
    one16 = jnp.ones((16,), jnp.float32)

    def fill_ones(i, _):
        ones_v[i, :] = one16
        return 0

    lax.fori_loop(0, 128, fill_ones, 0)

    base = s * ROWS_PER_TILE
    for k in range(ROWS_PER_TILE // 64):
        pltpu.sync_copy(zbuf_v, acc_sh.at[pl.ds(base + k * 64, 64)])
    pltpu.sync_copy(dst_hbm.at[wid], dst_v)
    plsc.subcore_barrier()

    def body(j, _):
        pltpu.sync_copy(ones_v, acc_sh.at[dst_v.at[j]], add=True)
        return 0

    lax.fori_loop(0, CHUNKS, body, 0)
    plsc.subcore_barrier()
    pltpu.sync_copy(acc_sh.at[pl.ds(base, ROWS_PER_TILE)],
                    out_hbm.at[c, pl.ds(base, ROWS_PER_TILE)])


def _sc_deg(dst3):
    mesh = plsc.VectorSubcoreMesh(core_axis_name="c", subcore_axis_name="s")
    return pl.kernel(
        _deg_kernel,
        mesh=mesh,
        compiler_params=pltpu.CompilerParams(use_tc_tiling_on_sc=False),
        out_type=jax.ShapeDtypeStruct((NC, NP, 16), jnp.float32),
        scratch_types=[
            pltpu.VMEM((CHUNKS, 128), jnp.int32),
            pltpu.VMEM((128, 16), jnp.float32),
            pltpu.VMEM((64, 16), jnp.float32),
            pltpu.VMEM_SHARED((NP, 16), jnp.float32),
        ],
    )(dst3)


def _dinv_body(degp_ref, o_ref):
    deg = degp_ref[0, :, 0:1] + degp_ref[1, :, 0:1] + 1.0
    o_ref[...] = lax.rsqrt(deg)


def _tc_dinv(deg_partials):
    return pl.pallas_call(
        _dinv_body,
        out_shape=jax.ShapeDtypeStruct((NP, 1), jnp.float32),
    )(deg_partials)


def _g0_body(x_ref, w_ref, dinv_ref, o_ref):
    o_ref[...] = dinv_ref[...] * jnp.dot(
        x_ref[...], w_ref[...], preferred_element_type=jnp.float32)


def _tc_g0(x, W0, dinv):
    blk = 512
    grid = NP // blk
    return pl.pallas_call(
        _g0_body,
        grid=(grid,),
        in_specs=[
            pl.BlockSpec((blk, x.shape[1]), lambda i: (i, 0)),
            pl.BlockSpec((x.shape[1], D), lambda i: (0, 0)),
            pl.BlockSpec((blk, 1), lambda i: (i, 0)),
        ],
        out_specs=pl.BlockSpec((blk, D), lambda i: (i, 0)),
        out_shape=jax.ShapeDtypeStruct((NP, D), jnp.float32),
    )(x, W0, dinv)


def _mid_body(s_ref, g_ref, dinv_ref, b_ref, w_ref, o_ref):
    dinv = dinv_ref[...]
    a = dinv * (s_ref[0] + s_ref[1] + g_ref[...]) + b_ref[...]
    r = jnp.maximum(a, 0.0)
    o_ref[...] = dinv * jnp.dot(r, w_ref[...],
                                preferred_element_type=jnp.float32)


def _tc_mid(S, g, dinv, b, W):
    blk = 512
    grid = NP // blk
    return pl.pallas_call(
        _mid_body,
        grid=(grid,),
        in_specs=[
            pl.BlockSpec((NC, blk, D), lambda i: (0, i, 0)),
            pl.BlockSpec((blk, D), lambda i: (i, 0)),
            pl.BlockSpec((blk, 1), lambda i: (i, 0)),
            pl.BlockSpec((1, D), lambda i: (0, 0)),
            pl.BlockSpec((D, D), lambda i: (0, 0)),
        ],
        out_specs=pl.BlockSpec((blk, D), lambda i: (i, 0)),
        out_shape=jax.ShapeDtypeStruct((NP, D), jnp.float32),
    )(S, g, dinv, b, W)


def _final_body(s_ref, g_ref, dinv_ref, b_ref, batch_ref, wlin_ref, blin_ref,
                o_ref, sums_ref, cnts_ref):
    i = pl.program_id(0)

    @pl.when(i == 0)
    def _():
        sums_ref[...] = jnp.zeros_like(sums_ref)
        cnts_ref[...] = jnp.zeros_like(cnts_ref)

    dinv = dinv_ref[...]
    a = dinv * (s_ref[0] + s_ref[1] + g_ref[...]) + b_ref[...]
    r = jnp.maximum(a, 0.0)
    bt = batch_ref[...].reshape(1, -1)
    gid = lax.broadcasted_iota(jnp.int32, (NUM_GRAPHS, bt.shape[1]), 0)
    oh = (gid == bt).astype(jnp.float32)
    sums_ref[...] += jnp.dot(oh, r, preferred_element_type=jnp.float32)
    cnts_ref[...] += jnp.sum(oh, axis=1, keepdims=True)

    @pl.when(i == pl.num_programs(0) - 1)
    def _():
        pooled = sums_ref[...] / jnp.maximum(cnts_ref[...], 1.0)
        o_ref[...] = jnp.dot(pooled, wlin_ref[...],
                             preferred_element_type=jnp.float32) + blin_ref[...]


def _tc_final(S, g, dinv, b, batch2, Wlin, blin):
    blk = 512
    grid = NP // blk
    return pl.pallas_call(
        _final_body,
        grid=(grid,),
        in_specs=[
            pl.BlockSpec((NC, blk, D), lambda i: (0, i, 0)),
            pl.BlockSpec((blk, D), lambda i: (i, 0)),
            pl.BlockSpec((blk, 1), lambda i: (i, 0)),
            pl.BlockSpec((1, D), lambda i: (0, 0)),
            pl.BlockSpec((blk, 1), lambda i: (i, 0)),
            pl.BlockSpec((D, Wlin.shape[1]), lambda i: (0, 0)),
            pl.BlockSpec((1, Wlin.shape[1]), lambda i: (0, 0)),
        ],
        out_specs=pl.BlockSpec((NUM_GRAPHS, Wlin.shape[1]), lambda i: (0, 0)),
        out_shape=jax.ShapeDtypeStruct((NUM_GRAPHS, Wlin.shape[1]), jnp.float32),
        scratch_shapes=[
            pltpu.VMEM((NUM_GRAPHS, D), jnp.float32),
            pltpu.VMEM((NUM_GRAPHS, 1), jnp.float32),
        ],
    )(S, g, dinv, b, batch2, Wlin, blin)


@jax.jit
def kernel(x, edge_index, batch, W0, b0, W1, b1, W2, b2, Wlin, blin):
    n = x.shape[0]
    xp = jnp.zeros((NP, x.shape[1]), x.dtype).at[:n].set(x)
    src = jnp.full((EP,), n, jnp.int32).at[:E].set(edge_index[0])
    dst = jnp.full((EP,), NP - 1, jnp.int32).at[:E].set(edge_index[1])
    src3 = src.reshape(NC * NS, CHUNKS, 128)
    dst3 = dst.reshape(NC * NS, CHUNKS, 128)
    batch2 = jnp.full((NP, 1), NUM_GRAPHS + 7, jnp.int32).at[:n, 0].set(batch)

    deg_partials = _sc_deg(dst3)
    dinv = _tc_dinv(deg_partials)

    g0 = _tc_g0(xp, W0, dinv)
    S0 = _sc_scatter(g0, src3, dst3)
    g1 = _tc_mid(S0, g0, dinv, b0.reshape(1, D), W1)
    S1 = _sc_scatter(g1, src3, dst3)
    g2 = _tc_mid(S1, g1, dinv, b1.reshape(1, D), W2)
    S2 = _sc_scatter(g2, src3, dst3)
    out = _tc_final(S2, g2, dinv, b2.reshape(1, D), batch2,
                    Wlin, blin.reshape(1, -1))
    return out


# speedup vs baseline: 28.7447x; 1.9177x over previous
import functools

import jax
import jax.numpy as jnp
from jax import lax
from jax.experimental import pallas as pl
from jax.experimental.pallas import tpu as pltpu
from jax.experimental.pallas import tpu_sc as plsc

N_NODES = 10000
NP = 10240
E = 320000
EP = 327680
D = 64
DH = D // 2
NUM_GRAPHS = 128
NC, NS = 2, 16
ROWS_PER_TILE = NP // NS
CHUNKS = EP // NS // 128
NBUF = 4


def _zero_fill(buf, n_rows, width):
    zero16 = jnp.zeros((16,), jnp.float32)
    cols = width // 16

    def body(i, _):
        r = i // cols
        c = (i % cols) * 16
        buf[r, pl.ds(c, 16)] = zero16
        return 0

    lax.fori_loop(0, n_rows * cols, body, 0)


def _scatter_kernel(g_hbm, src_hbm, dst_hbm, out_hbm, src_v, dst_v, rows_v,
                    zbuf_v, g_sh, acc_sh, sem):
    c = lax.axis_index("c")
    s = lax.axis_index("s")

    _zero_fill(zbuf_v, 64, DH)
    base = s * ROWS_PER_TILE
    pltpu.sync_copy(g_hbm.at[c, pl.ds(base, ROWS_PER_TILE)],
                    g_sh.at[pl.ds(base, ROWS_PER_TILE)])
    for k in range(ROWS_PER_TILE // 64):
        pltpu.sync_copy(zbuf_v, acc_sh.at[pl.ds(base + k * 64, 64)])

    pltpu.sync_copy(src_hbm.at[s], src_v)
    pltpu.sync_copy(dst_hbm.at[s], dst_v)
    plsc.subcore_barrier()

    for b in range(NBUF):
        pltpu.async_copy(g_sh.at[src_v.at[b]], rows_v.at[b], sem[b])

    def body(g, _):
        for b in range(NBUF):
            j = g * NBUF + b
            pltpu.make_async_copy(g_sh.at[src_v.at[j]], rows_v.at[b],
                                  sem[b]).wait()
            pltpu.sync_copy(rows_v.at[b], acc_sh.at[dst_v.at[j]], add=True)

            @pl.when(j + NBUF < CHUNKS)
            def _():
                pltpu.async_copy(g_sh.at[src_v.at[j + NBUF]], rows_v.at[b],
                                 sem[b])
        return 0

    lax.fori_loop(0, CHUNKS // NBUF, body, 0)
    plsc.subcore_barrier()

    pltpu.sync_copy(acc_sh.at[pl.ds(base, ROWS_PER_TILE)],
                    out_hbm.at[c, pl.ds(base, ROWS_PER_TILE)])


def _sc_scatter(g, src3, dst3):
    mesh = plsc.VectorSubcoreMesh(core_axis_name="c", subcore_axis_name="s")
    return pl.kernel(
        _scatter_kernel,
        mesh=mesh,
        compiler_params=pltpu.CompilerParams(use_tc_tiling_on_sc=False),
        out_type=jax.ShapeDtypeStruct((NC, NP, DH), jnp.float32),
        scratch_types=[
            pltpu.VMEM((CHUNKS, 128), jnp.int32),
            pltpu.VMEM((CHUNKS, 128), jnp.int32),
            pltpu.VMEM((NBUF, 128, DH), jnp.float32),
            pltpu.VMEM((64, DH), jnp.float32),
            pltpu.VMEM_SHARED((NP, DH), jnp.float32),
            pltpu.VMEM_SHARED((NP, DH), jnp.float32),
            [pltpu.SemaphoreType.DMA] * NBUF,
        ],
    )(g, src3, dst3)


def _deg_kernel(dst_hbm, out_hbm, dst_v, ones_v, zbuf_v, acc_sh):
    c = lax.axis_index("c")
    s = lax.axis_index("s")

    _zero_fill(zbuf_v, 64, 16)
    one16 = jnp.ones((16,), jnp.float32)

    def fill_ones(i, _):
        ones_v[i, :] = one16
        return 0

    lax.fori_loop(0, 128, fill_ones, 0)

    base = s * ROWS_PER_TILE
    for k in range(ROWS_PER_TILE // 64):
        pltpu.sync_copy(zbuf_v, acc_sh.at[pl.ds(base + k * 64, 64)])
    pltpu.sync_copy(dst_hbm.at[s], dst_v)
    plsc.subcore_barrier()

    half = CHUNKS // 2

    def body(j, _):
        pltpu.sync_copy(ones_v, acc_sh.at[dst_v.at[c * half + j]], add=True)
        return 0

    lax.fori_loop(0, half, body, 0)
    plsc.subcore_barrier()
    pltpu.sync_copy(acc_sh.at[pl.ds(base, ROWS_PER_TILE)],
                    out_hbm.at[c, pl.ds(base, ROWS_PER_TILE)])


def _sc_deg(dst3):
    mesh = plsc.VectorSubcoreMesh(core_axis_name="c", subcore_axis_name="s")
    return pl.kernel(
        _deg_kernel,
        mesh=mesh,
        compiler_params=pltpu.CompilerParams(use_tc_tiling_on_sc=False),
        out_type=jax.ShapeDtypeStruct((NC, NP, 16), jnp.float32),
        scratch_types=[
            pltpu.VMEM((CHUNKS, 128), jnp.int32),
            pltpu.VMEM((128, 16), jnp.float32),
            pltpu.VMEM((64, 16), jnp.float32),
            pltpu.VMEM_SHARED((NP, 16), jnp.float32),
        ],
    )(dst3)


def _dinv_body(degp_ref, o_ref):
    deg = degp_ref[0, :, 0:1] + degp_ref[1, :, 0:1] + 1.0
    o_ref[...] = lax.rsqrt(deg)


def _tc_dinv(deg_partials):
    return pl.pallas_call(
        _dinv_body,
        out_shape=jax.ShapeDtypeStruct((NP, 1), jnp.float32),
    )(deg_partials)


def _split(res, o_ref):
    o_ref[0] = res[:, :DH]
    o_ref[1] = res[:, DH:]


def _g0_body(x_ref, w_ref, dinv_ref, o_ref):
    _split(dinv_ref[...] * jnp.dot(x_ref[...], w_ref[...],
                                   preferred_element_type=jnp.float32), o_ref)


def _tc_g0(x, W0, dinv):
    blk = 512
    grid = NP // blk
    return pl.pallas_call(
        _g0_body,
        grid=(grid,),
        in_specs=[
            pl.BlockSpec((blk, x.shape[1]), lambda i: (i, 0)),
            pl.BlockSpec((x.shape[1], D), lambda i: (0, 0)),
            pl.BlockSpec((blk, 1), lambda i: (i, 0)),
        ],
        out_specs=pl.BlockSpec((NC, blk, DH), lambda i: (0, i, 0)),
        out_shape=jax.ShapeDtypeStruct((NC, NP, DH), jnp.float32),
    )(x, W0, dinv)


def _pre_relu(s_ref, g_ref, dinv_ref, b_ref):
    sg = jnp.concatenate([s_ref[0] + g_ref[0], s_ref[1] + g_ref[1]], axis=1)
    a = dinv_ref[...] * sg + b_ref[...]
    return jnp.maximum(a, 0.0)


def _mid_body(s_ref, g_ref, dinv_ref, b_ref, w_ref, o_ref):
    r = _pre_relu(s_ref, g_ref, dinv_ref, b_ref)
    _split(dinv_ref[...] * jnp.dot(r, w_ref[...],
                                   preferred_element_type=jnp.float32), o_ref)


def _tc_mid(S, g, dinv, b, W):
    blk = 512
    grid = NP // blk
    return pl.pallas_call(
        _mid_body,
        grid=(grid,),
        in_specs=[
            pl.BlockSpec((NC, blk, DH), lambda i: (0, i, 0)),
            pl.BlockSpec((NC, blk, DH), lambda i: (0, i, 0)),
            pl.BlockSpec((blk, 1), lambda i: (i, 0)),
            pl.BlockSpec((1, D), lambda i: (0, 0)),
            pl.BlockSpec((D, D), lambda i: (0, 0)),
        ],
        out_specs=pl.BlockSpec((NC, blk, DH), lambda i: (0, i, 0)),
        out_shape=jax.ShapeDtypeStruct((NC, NP, DH), jnp.float32),
    )(S, g, dinv, b, W)


def _final_body(s_ref, g_ref, dinv_ref, b_ref, batch_ref, wlin_ref, blin_ref,
                o_ref, sums_ref, cnts_ref):
    i = pl.program_id(0)

    @pl.when(i == 0)
    def _():
        sums_ref[...] = jnp.zeros_like(sums_ref)
        cnts_ref[...] = jnp.zeros_like(cnts_ref)

    r = _pre_relu(s_ref, g_ref, dinv_ref, b_ref)
    bt = batch_ref[...].reshape(1, -1)
    gid = lax.broadcasted_iota(jnp.int32, (NUM_GRAPHS, bt.shape[1]), 0)
    oh = (gid == bt).astype(jnp.float32)
    sums_ref[...] += jnp.dot(oh, r, preferred_element_type=jnp.float32)
    cnts_ref[...] += jnp.sum(oh, axis=1, keepdims=True)

    @pl.when(i == pl.num_programs(0) - 1)
    def _():
        pooled = sums_ref[...] / jnp.maximum(cnts_ref[...], 1.0)
        o_ref[...] = jnp.dot(pooled, wlin_ref[...],
                             preferred_element_type=jnp.float32) + blin_ref[...]


def _tc_final(S, g, dinv, b, batch2, Wlin, blin):
    blk = 512
    grid = NP // blk
    return pl.pallas_call(
        _final_body,
        grid=(grid,),
        in_specs=[
            pl.BlockSpec((NC, blk, DH), lambda i: (0, i, 0)),
            pl.BlockSpec((NC, blk, DH), lambda i: (0, i, 0)),
            pl.BlockSpec((blk, 1), lambda i: (i, 0)),
            pl.BlockSpec((1, D), lambda i: (0, 0)),
            pl.BlockSpec((blk, 1), lambda i: (i, 0)),
            pl.BlockSpec((D, Wlin.shape[1]), lambda i: (0, 0)),
            pl.BlockSpec((1, Wlin.shape[1]), lambda i: (0, 0)),
        ],
        out_specs=pl.BlockSpec((NUM_GRAPHS, Wlin.shape[1]), lambda i: (0, 0)),
        out_shape=jax.ShapeDtypeStruct((NUM_GRAPHS, Wlin.shape[1]), jnp.float32),
        scratch_shapes=[
            pltpu.VMEM((NUM_GRAPHS, D), jnp.float32),
            pltpu.VMEM((NUM_GRAPHS, 1), jnp.float32),
        ],
    )(S, g, dinv, b, batch2, Wlin, blin)


@jax.jit
def kernel(x, edge_index, batch, W0, b0, W1, b1, W2, b2, Wlin, blin):
    n = x.shape[0]
    xp = jnp.zeros((NP, x.shape[1]), x.dtype).at[:n].set(x)
    src = jnp.full((EP,), n, jnp.int32).at[:E].set(edge_index[0])
    dst = jnp.full((EP,), NP - 1, jnp.int32).at[:E].set(edge_index[1])
    src3 = src.reshape(NS, CHUNKS, 128)
    dst3 = dst.reshape(NS, CHUNKS, 128)
    batch2 = jnp.full((NP, 1), NUM_GRAPHS + 7, jnp.int32).at[:n, 0].set(batch)

    deg_partials = _sc_deg(dst3)
    dinv = _tc_dinv(deg_partials)

    g0 = _tc_g0(xp, W0, dinv)
    S0 = _sc_scatter(g0, src3, dst3)
    g1 = _tc_mid(S0, g0, dinv, b0.reshape(1, D), W1)
    S1 = _sc_scatter(g1, src3, dst3)
    g2 = _tc_mid(S1, g1, dinv, b1.reshape(1, D), W2)
    S2 = _sc_scatter(g2, src3, dst3)
    out = _tc_final(S2, g2, dinv, b2.reshape(1, D), batch2,
                    Wlin, blin.reshape(1, -1))
    return out


# speedup vs baseline: 31.3410x; 1.0903x over previous
import functools

import jax
import jax.numpy as jnp
from jax import lax
from jax.experimental import pallas as pl
from jax.experimental.pallas import tpu as pltpu
from jax.experimental.pallas import tpu_sc as plsc

N_NODES = 10000
NP = 10240
E = 320000
EP = 327680
D = 64
DH = D // 2
NUM_GRAPHS = 128
NC, NS = 2, 16
ROWS_PER_TILE = NP // NS
CHUNKS = EP // NS // 128
NBUF = 4


def _zero_fill(buf, n_rows, width):
    zero16 = jnp.zeros((16,), jnp.float32)
    cols = width // 16

    def body(i, _):
        r = i // cols
        c = (i % cols) * 16
        buf[r, pl.ds(c, 16)] = zero16
        return 0

    lax.fori_loop(0, n_rows * cols, body, 0)


def _scatter_kernel(g_hbm, src_hbm, dst_hbm, out_hbm, src_v, dst_v, rows_v,
                    zbuf_v, g_sh, acc_sh, sem):
    c = lax.axis_index("c")
    s = lax.axis_index("s")

    _zero_fill(zbuf_v, 64, DH)
    base = s * ROWS_PER_TILE
    pltpu.sync_copy(g_hbm.at[c, pl.ds(base, ROWS_PER_TILE)],
                    g_sh.at[pl.ds(base, ROWS_PER_TILE)])
    for k in range(ROWS_PER_TILE // 64):
        pltpu.sync_copy(zbuf_v, acc_sh.at[pl.ds(base + k * 64, 64)])

    pltpu.sync_copy(src_hbm.at[s], src_v)
    pltpu.sync_copy(dst_hbm.at[s], dst_v)
    plsc.subcore_barrier()

    for b in range(NBUF):
        pltpu.async_copy(g_sh.at[src_v.at[b]], rows_v.at[b], sem[b])

    def body(g, _):
        for b in range(NBUF):
            j = g * NBUF + b
            pltpu.make_async_copy(g_sh.at[src_v.at[j]], rows_v.at[b],
                                  sem[b]).wait()
            pltpu.sync_copy(rows_v.at[b], acc_sh.at[dst_v.at[j]], add=True)

            @pl.when(j + NBUF < CHUNKS)
            def _():
                pltpu.async_copy(g_sh.at[src_v.at[j + NBUF]], rows_v.at[b],
                                 sem[b])
        return 0

    lax.fori_loop(0, CHUNKS // NBUF, body, 0)
    plsc.subcore_barrier()

    pltpu.sync_copy(acc_sh.at[pl.ds(base, ROWS_PER_TILE)],
                    out_hbm.at[c, pl.ds(base, ROWS_PER_TILE)])


def _sc_scatter(g, src3, dst3):
    mesh = plsc.VectorSubcoreMesh(core_axis_name="c", subcore_axis_name="s")
    return pl.kernel(
        _scatter_kernel,
        mesh=mesh,
        compiler_params=pltpu.CompilerParams(use_tc_tiling_on_sc=False),
        out_type=jax.ShapeDtypeStruct((NC, NP, DH), jnp.float32),
        scratch_types=[
            pltpu.VMEM((CHUNKS, 128), jnp.int32),
            pltpu.VMEM((CHUNKS, 128), jnp.int32),
            pltpu.VMEM((NBUF, 128, DH), jnp.float32),
            pltpu.VMEM((64, DH), jnp.float32),
            pltpu.VMEM_SHARED((NP, DH), jnp.float32),
            pltpu.VMEM_SHARED((NP, DH), jnp.float32),
            [pltpu.SemaphoreType.DMA] * NBUF,
        ],
    )(g, src3, dst3)


def _deg_kernel(dst_hbm, out_hbm, dst_v, ones_v, zbuf_v, acc_sh):
    c = lax.axis_index("c")
    s = lax.axis_index("s")

    _zero_fill(zbuf_v, 64, 16)
    one16 = jnp.ones((16,), jnp.float32)

    def fill_ones(i, _):
        ones_v[i, :] = one16
        return 0

    lax.fori_loop(0, 128, fill_ones, 0)

    base = s * ROWS_PER_TILE
    for k in range(ROWS_PER_TILE // 64):
        pltpu.sync_copy(zbuf_v, acc_sh.at[pl.ds(base + k * 64, 64)])
    pltpu.sync_copy(dst_hbm.at[s], dst_v)
    plsc.subcore_barrier()

    half = CHUNKS // 2

    def body(j, _):
        pltpu.sync_copy(ones_v, acc_sh.at[dst_v.at[c * half + j]], add=True)
        return 0

    lax.fori_loop(0, half, body, 0)
    plsc.subcore_barrier()
    pltpu.sync_copy(acc_sh.at[pl.ds(base, ROWS_PER_TILE)],
                    out_hbm.at[c, pl.ds(base, ROWS_PER_TILE)])


def _sc_deg(dst3):
    mesh = plsc.VectorSubcoreMesh(core_axis_name="c", subcore_axis_name="s")
    return pl.kernel(
        _deg_kernel,
        mesh=mesh,
        compiler_params=pltpu.CompilerParams(use_tc_tiling_on_sc=False),
        out_type=jax.ShapeDtypeStruct((NC, NP, 16), jnp.float32),
        scratch_types=[
            pltpu.VMEM((CHUNKS, 128), jnp.int32),
            pltpu.VMEM((128, 16), jnp.float32),
            pltpu.VMEM((64, 16), jnp.float32),
            pltpu.VMEM_SHARED((NP, 16), jnp.float32),
        ],
    )(dst3)


def _split(res, o_ref):
    o_ref[0] = res[:, :DH]
    o_ref[1] = res[:, DH:]


def _g0_body(x_ref, w_ref, degp_ref, o_ref, dinv_ref):
    deg = degp_ref[0, :, 0:1] + degp_ref[1, :, 0:1] + 1.0
    dinv = lax.rsqrt(deg)
    dinv_ref[...] = dinv
    _split(dinv * jnp.dot(x_ref[...], w_ref[...],
                          preferred_element_type=jnp.float32), o_ref)


def _tc_g0(x, W0, degp):
    blk = 2048
    grid = NP // blk
    return pl.pallas_call(
        _g0_body,
        grid=(grid,),
        in_specs=[
            pl.BlockSpec((blk, x.shape[1]), lambda i: (i, 0)),
            pl.BlockSpec((x.shape[1], D), lambda i: (0, 0)),
            pl.BlockSpec((NC, blk, 16), lambda i: (0, i, 0)),
        ],
        out_specs=[
            pl.BlockSpec((NC, blk, DH), lambda i: (0, i, 0)),
            pl.BlockSpec((blk, 1), lambda i: (i, 0)),
        ],
        out_shape=[
            jax.ShapeDtypeStruct((NC, NP, DH), jnp.float32),
            jax.ShapeDtypeStruct((NP, 1), jnp.float32),
        ],
    )(x, W0, degp)


def _pre_relu(s_ref, g_ref, dinv_ref, b_ref):
    sg = jnp.concatenate([s_ref[0] + g_ref[0], s_ref[1] + g_ref[1]], axis=1)
    a = dinv_ref[...] * sg + b_ref[...]
    return jnp.maximum(a, 0.0)


def _mid_body(s_ref, g_ref, dinv_ref, b_ref, w_ref, o_ref):
    r = _pre_relu(s_ref, g_ref, dinv_ref, b_ref)
    _split(dinv_ref[...] * jnp.dot(r, w_ref[...],
                                   preferred_element_type=jnp.float32), o_ref)


def _tc_mid(S, g, dinv, b, W):
    blk = 2048
    grid = NP // blk
    return pl.pallas_call(
        _mid_body,
        grid=(grid,),
        in_specs=[
            pl.BlockSpec((NC, blk, DH), lambda i: (0, i, 0)),
            pl.BlockSpec((NC, blk, DH), lambda i: (0, i, 0)),
            pl.BlockSpec((blk, 1), lambda i: (i, 0)),
            pl.BlockSpec((1, D), lambda i: (0, 0)),
            pl.BlockSpec((D, D), lambda i: (0, 0)),
        ],
        out_specs=pl.BlockSpec((NC, blk, DH), lambda i: (0, i, 0)),
        out_shape=jax.ShapeDtypeStruct((NC, NP, DH), jnp.float32),
    )(S, g, dinv, b, W)


def _final_body(s_ref, g_ref, dinv_ref, b_ref, batch_ref, wlin_ref, blin_ref,
                o_ref, sums_ref, cnts_ref):
    i = pl.program_id(0)

    @pl.when(i == 0)
    def _():
        sums_ref[...] = jnp.zeros_like(sums_ref)
        cnts_ref[...] = jnp.zeros_like(cnts_ref)

    r = _pre_relu(s_ref, g_ref, dinv_ref, b_ref)
    bt = batch_ref[...].reshape(1, -1)
    gid = lax.broadcasted_iota(jnp.int32, (NUM_GRAPHS, bt.shape[1]), 0)
    oh = (gid == bt).astype(jnp.float32)
    sums_ref[...] += jnp.dot(oh, r, preferred_element_type=jnp.float32)
    cnts_ref[...] += jnp.sum(oh, axis=1, keepdims=True)

    @pl.when(i == pl.num_programs(0) - 1)
    def _():
        pooled = sums_ref[...] / jnp.maximum(cnts_ref[...], 1.0)
        o_ref[...] = jnp.dot(pooled, wlin_ref[...],
                             preferred_element_type=jnp.float32) + blin_ref[...]


def _tc_final(S, g, dinv, b, batch2, Wlin, blin):
    blk = 2048
    grid = NP // blk
    return pl.pallas_call(
        _final_body,
        grid=(grid,),
        in_specs=[
            pl.BlockSpec((NC, blk, DH), lambda i: (0, i, 0)),
            pl.BlockSpec((NC, blk, DH), lambda i: (0, i, 0)),
            pl.BlockSpec((blk, 1), lambda i: (i, 0)),
            pl.BlockSpec((1, D), lambda i: (0, 0)),
            pl.BlockSpec((blk, 1), lambda i: (i, 0)),
            pl.BlockSpec((D, Wlin.shape[1]), lambda i: (0, 0)),
            pl.BlockSpec((1, Wlin.shape[1]), lambda i: (0, 0)),
        ],
        out_specs=pl.BlockSpec((NUM_GRAPHS, Wlin.shape[1]), lambda i: (0, 0)),
        out_shape=jax.ShapeDtypeStruct((NUM_GRAPHS, Wlin.shape[1]), jnp.float32),
        scratch_shapes=[
            pltpu.VMEM((NUM_GRAPHS, D), jnp.float32),
            pltpu.VMEM((NUM_GRAPHS, 1), jnp.float32),
        ],
    )(S, g, dinv, b, batch2, Wlin, blin)


@jax.jit
def kernel(x, edge_index, batch, W0, b0, W1, b1, W2, b2, Wlin, blin):
    n = x.shape[0]
    xp = jnp.zeros((NP, x.shape[1]), x.dtype).at[:n].set(x)
    src = jnp.full((EP,), n, jnp.int32).at[:E].set(edge_index[0])
    dst = jnp.full((EP,), NP - 1, jnp.int32).at[:E].set(edge_index[1])
    src3 = src.reshape(NS, CHUNKS, 128)
    dst3 = dst.reshape(NS, CHUNKS, 128)
    batch2 = jnp.full((NP, 1), NUM_GRAPHS + 7, jnp.int32).at[:n, 0].set(batch)

    deg_partials = _sc_deg(dst3)
    g0, dinv = _tc_g0(xp, W0, deg_partials)
    S0 = _sc_scatter(g0, src3, dst3)
    g1 = _tc_mid(S0, g0, dinv, b0.reshape(1, D), W1)
    S1 = _sc_scatter(g1, src3, dst3)
    g2 = _tc_mid(S1, g1, dinv, b1.reshape(1, D), W2)
    S2 = _sc_scatter(g2, src3, dst3)
    out = _tc_final(S2, g2, dinv, b2.reshape(1, D), batch2,
                    Wlin, blin.reshape(1, -1))
    return out


# speedup vs baseline: 39.5612x; 1.2623x over previous
import functools

import jax
import jax.numpy as jnp
from jax import lax
from jax.experimental import pallas as pl
from jax.experimental.pallas import tpu as pltpu
from jax.experimental.pallas import tpu_sc as plsc

N_NODES = 10000
NP = 10240
E = 320000
EP = 327680
D = 64
DH = D // 2
NUM_GRAPHS = 128
NC, NS = 2, 16
ROWS_PER_TILE = NP // NS
CHUNKS = EP // NS // 128
NBUF = 4


def _zero_fill(buf, n_rows, width):
    zero16 = jnp.zeros((16,), jnp.float32)
    cols = width // 16

    def body(i, _):
        r = i // cols
        c = (i % cols) * 16
        buf[r, pl.ds(c, 16)] = zero16
        return 0

    lax.fori_loop(0, n_rows * cols, body, 0)


def _scatter_kernel(g_hbm, src_hbm, dst_hbm, out_hbm, src_v, dst_v, rows_v,
                    zbuf_v, g_sh, acc_sh, sem):
    c = lax.axis_index("c")
    s = lax.axis_index("s")

    _zero_fill(zbuf_v, 64, DH)
    base = s * ROWS_PER_TILE
    pltpu.sync_copy(g_hbm.at[c, pl.ds(base, ROWS_PER_TILE)],
                    g_sh.at[pl.ds(base, ROWS_PER_TILE)])
    for k in range(ROWS_PER_TILE // 64):
        pltpu.sync_copy(zbuf_v, acc_sh.at[pl.ds(base + k * 64, 64)])

    pltpu.sync_copy(src_hbm.at[s], src_v)
    pltpu.sync_copy(dst_hbm.at[s], dst_v)
    plsc.subcore_barrier()

    for b in range(NBUF):
        pltpu.async_copy(g_sh.at[src_v.at[b]], rows_v.at[b], sem[b])

    def body(g, _):
        for b in range(NBUF):
            j = g * NBUF + b
            pltpu.make_async_copy(g_sh.at[src_v.at[j]], rows_v.at[b],
                                  sem[b]).wait()
            pltpu.sync_copy(rows_v.at[b], acc_sh.at[dst_v.at[j]], add=True)

            @pl.when(j + NBUF < CHUNKS)
            def _():
                pltpu.async_copy(g_sh.at[src_v.at[j + NBUF]], rows_v.at[b],
                                 sem[b])
        return 0

    lax.fori_loop(0, CHUNKS // NBUF, body, 0)
    plsc.subcore_barrier()

    pltpu.sync_copy(acc_sh.at[pl.ds(base, ROWS_PER_TILE)],
                    out_hbm.at[c, pl.ds(base, ROWS_PER_TILE)])


def _sc_scatter(g, src3, dst3):
    mesh = plsc.VectorSubcoreMesh(core_axis_name="c", subcore_axis_name="s")
    return pl.kernel(
        _scatter_kernel,
        mesh=mesh,
        compiler_params=pltpu.CompilerParams(use_tc_tiling_on_sc=False),
        out_type=jax.ShapeDtypeStruct((NC, NP, DH), jnp.float32),
        scratch_types=[
            pltpu.VMEM((CHUNKS, 128), jnp.int32),
            pltpu.VMEM((CHUNKS, 128), jnp.int32),
            pltpu.VMEM((NBUF, 128, DH), jnp.float32),
            pltpu.VMEM((64, DH), jnp.float32),
            pltpu.VMEM_SHARED((NP, DH), jnp.float32),
            pltpu.VMEM_SHARED((NP, DH), jnp.float32),
            [pltpu.SemaphoreType.DMA] * NBUF,
        ],
    )(g, src3, dst3)


def _deg_kernel(dst_hbm, out_hbm, dst_v, ones_v, zbuf_v, acc_sh):
    c = lax.axis_index("c")
    s = lax.axis_index("s")

    _zero_fill(zbuf_v, 64, 32)
    one16 = jnp.ones((16,), jnp.float32)

    def fill_ones(i, _):
        ones_v[i // 2, pl.ds((i % 2) * 16, 16)] = one16
        return 0

    lax.fori_loop(0, 256, fill_ones, 0)

    base = s * ROWS_PER_TILE
    for k in range(ROWS_PER_TILE // 64):
        pltpu.sync_copy(zbuf_v, acc_sh.at[pl.ds(base + k * 64, 64)])
    pltpu.sync_copy(dst_hbm.at[s], dst_v)
    plsc.subcore_barrier()

    half = CHUNKS // 2

    def body(j, _):
        pltpu.sync_copy(ones_v, acc_sh.at[dst_v.at[c * half + j]], add=True)
        return 0

    lax.fori_loop(0, half, body, 0)
    plsc.subcore_barrier()
    pltpu.sync_copy(acc_sh.at[pl.ds(base, ROWS_PER_TILE)],
                    out_hbm.at[c, pl.ds(base, ROWS_PER_TILE)])


def _sc_deg(dst3):
    mesh = plsc.VectorSubcoreMesh(core_axis_name="c", subcore_axis_name="s")
    return pl.kernel(
        _deg_kernel,
        mesh=mesh,
        compiler_params=pltpu.CompilerParams(use_tc_tiling_on_sc=False),
        out_type=jax.ShapeDtypeStruct((NC, NP, 32), jnp.float32),
        scratch_types=[
            pltpu.VMEM((CHUNKS, 128), jnp.int32),
            pltpu.VMEM((128, 32), jnp.float32),
            pltpu.VMEM((64, 32), jnp.float32),
            pltpu.VMEM_SHARED((NP, 32), jnp.float32),
        ],
    )(dst3)


NR = NP // 4


def _kron4(wsub, rows):
    t = jnp.concatenate([wsub] * 4, axis=0)
    t = jnp.concatenate([t] * 4, axis=1)
    ri = lax.broadcasted_iota(jnp.int32, t.shape, 0) // (rows // 4)
    ci = lax.broadcasted_iota(jnp.int32, t.shape, 1) // 32
    return jnp.where(ri == ci, t, 0.0)


def _dinv_packed(degp_ref):
    deg = degp_ref[0] + degp_ref[1] + 1.0
    return lax.rsqrt(deg)


def _g0_body(x_ref, w_ref, degp_ref, o_ref):
    dinv = _dinv_packed(degp_ref)
    x2 = x_ref[...]
    for h in range(2):
        bd = _kron4(w_ref[:, h * DH:(h + 1) * DH], 512)
        o_ref[h] = dinv * jnp.dot(x2, bd, preferred_element_type=jnp.float32)


def _tc_g0(x2, W0, degp):
    return pl.pallas_call(
        _g0_body,
        out_shape=jax.ShapeDtypeStruct((NC, NR, 128), jnp.float32),
    )(x2, W0, degp)


def _relu_halves(s_ref, g_ref, dinv, b_ref):
    rs = []
    for h in range(2):
        bh = jnp.concatenate([b_ref[:, h * DH:(h + 1) * DH]] * 4, axis=1)
        rs.append(jnp.maximum(dinv * (s_ref[h] + g_ref[h]) + bh, 0.0))
    return rs


def _mid_body(s_ref, g_ref, degp_ref, b_ref, w_ref, o_ref):
    dinv = _dinv_packed(degp_ref)
    r = _relu_halves(s_ref, g_ref, dinv, b_ref)
    for h in range(2):
        acc = jnp.zeros((NR, 128), jnp.float32)
        for i in range(2):
            bd = _kron4(w_ref[i * DH:(i + 1) * DH, h * DH:(h + 1) * DH], 128)
            acc += jnp.dot(r[i], bd, preferred_element_type=jnp.float32)
        o_ref[h] = dinv * acc


def _tc_mid(S, g, degp, b, W):
    return pl.pallas_call(
        _mid_body,
        out_shape=jax.ShapeDtypeStruct((NC, NR, 128), jnp.float32),
    )(S, g, degp, b, W)


def _final_body(s_ref, g_ref, degp_ref, b_ref, batchp_ref, wlin_ref, blin_ref,
                o_ref):
    dinv = _dinv_packed(degp_ref)
    r = _relu_halves(s_ref, g_ref, dinv, b_ref)
    gid = lax.broadcasted_iota(jnp.int32, (NUM_GRAPHS, NR), 0)
    sums = []
    cnts = jnp.zeros((NUM_GRAPHS, 1), jnp.float32)
    for k in range(4):
        oh = (gid == batchp_ref[k:k + 1, :]).astype(jnp.float32)
        sums.append([jnp.dot(oh, r[h][:, k * DH:(k + 1) * DH],
                             preferred_element_type=jnp.float32)
                     for h in range(2)])
        cnts += jnp.sum(oh, axis=1, keepdims=True)
    pooled = jnp.concatenate(
        [sums[0][0] + sums[1][0] + sums[2][0] + sums[3][0],
         sums[0][1] + sums[1][1] + sums[2][1] + sums[3][1]],
        axis=1) / jnp.maximum(cnts, 1.0)
    o_ref[...] = jnp.dot(pooled, wlin_ref[...],
                         preferred_element_type=jnp.float32) + blin_ref[...]


def _tc_final(S, g, degp, b, batchp, Wlin, blin):
    return pl.pallas_call(
        _final_body,
        out_shape=jax.ShapeDtypeStruct((NUM_GRAPHS, Wlin.shape[1]),
                                       jnp.float32),
    )(S, g, degp, b, batchp, Wlin, blin)


@jax.jit
def kernel(x, edge_index, batch, W0, b0, W1, b1, W2, b2, Wlin, blin):
    n = x.shape[0]
    xp = jnp.zeros((NP, x.shape[1]), x.dtype).at[:n].set(x)
    src = jnp.full((EP,), n, jnp.int32).at[:E].set(edge_index[0])
    dst = jnp.full((EP,), NP - 1, jnp.int32).at[:E].set(edge_index[1])
    src3 = src.reshape(NS, CHUNKS, 128)
    dst3 = dst.reshape(NS, CHUNKS, 128)
    bb = jnp.full((NP,), NUM_GRAPHS + 7, jnp.int32).at[:n].set(batch)
    batchp = bb.reshape(NR, 4).T

    def to_sc(a):
        return a.reshape(NC, NP, DH)

    def to_tc(a):
        return a.reshape(NC, NR, 128)

    degp = to_tc(_sc_deg(dst3))
    g0 = _tc_g0(xp.reshape(NR, 512), W0, degp)
    S0 = to_tc(_sc_scatter(to_sc(g0), src3, dst3))
    g1 = _tc_mid(S0, g0, degp, b0.reshape(1, D), W1)
    S1 = to_tc(_sc_scatter(to_sc(g1), src3, dst3))
    g2 = _tc_mid(S1, g1, degp, b1.reshape(1, D), W2)
    S2 = to_tc(_sc_scatter(to_sc(g2), src3, dst3))
    out = _tc_final(S2, g2, degp, b2.reshape(1, D), batchp,
                    Wlin, blin.reshape(1, -1))
    return out


# speedup vs baseline: 41.2735x; 1.0433x over previous
import functools

import jax
import jax.numpy as jnp
from jax import lax
from jax.experimental import pallas as pl
from jax.experimental.pallas import tpu as pltpu
from jax.experimental.pallas import tpu_sc as plsc

N_NODES = 10000
NP = 10240
E = 320000
EP = 327680
D = 64
DH = D // 2
NUM_GRAPHS = 128
NC, NS = 2, 16
ROWS_PER_TILE = NP // NS
CHUNKS = EP // NS // 128
NBUF = 4


def _zero_fill(buf, n_rows, width):
    zero16 = jnp.zeros((16,), jnp.float32)
    cols = width // 16

    def body(i, _):
        r = i // cols
        c = (i % cols) * 16
        buf[r, pl.ds(c, 16)] = zero16
        return 0

    lax.fori_loop(0, n_rows * cols, body, 0)


def _scatter_kernel(g_hbm, src_hbm, dst_hbm, out_hbm, src_v, dst_v, rows_v,
                    zbuf_v, g_sh, acc_sh, sem):
    c = lax.axis_index("c")
    s = lax.axis_index("s")

    _zero_fill(zbuf_v, 64, DH)
    base = s * ROWS_PER_TILE
    pltpu.sync_copy(g_hbm.at[c, pl.ds(base, ROWS_PER_TILE)],
                    g_sh.at[pl.ds(base, ROWS_PER_TILE)])
    for k in range(ROWS_PER_TILE // 64):
        pltpu.sync_copy(zbuf_v, acc_sh.at[pl.ds(base + k * 64, 64)])

    pltpu.sync_copy(src_hbm.at[s], src_v)
    pltpu.sync_copy(dst_hbm.at[s], dst_v)
    plsc.subcore_barrier()

    for b in range(NBUF):
        pltpu.async_copy(g_sh.at[src_v.at[b]], rows_v.at[b], sem[b])

    def body(g, _):
        for b in range(NBUF):
            j = g * NBUF + b
            pltpu.make_async_copy(g_sh.at[src_v.at[j]], rows_v.at[b],
                                  sem[b]).wait()
            pltpu.sync_copy(rows_v.at[b], acc_sh.at[dst_v.at[j]], add=True)

            @pl.when(j + NBUF < CHUNKS)
            def _():
                pltpu.async_copy(g_sh.at[src_v.at[j + NBUF]], rows_v.at[b],
                                 sem[b])
        return 0

    lax.fori_loop(0, CHUNKS // NBUF, body, 0)
    plsc.subcore_barrier()

    pltpu.sync_copy(acc_sh.at[pl.ds(base, ROWS_PER_TILE)],
                    out_hbm.at[c, pl.ds(base, ROWS_PER_TILE)])


def _sc_scatter(g, src3, dst3):
    mesh = plsc.VectorSubcoreMesh(core_axis_name="c", subcore_axis_name="s")
    return pl.kernel(
        _scatter_kernel,
        mesh=mesh,
        compiler_params=pltpu.CompilerParams(use_tc_tiling_on_sc=False),
        out_type=jax.ShapeDtypeStruct((NC, NP, DH), jnp.float32),
        scratch_types=[
            pltpu.VMEM((CHUNKS, 128), jnp.int32),
            pltpu.VMEM((CHUNKS, 128), jnp.int32),
            pltpu.VMEM((NBUF, 128, DH), jnp.float32),
            pltpu.VMEM((64, DH), jnp.float32),
            pltpu.VMEM_SHARED((NP, DH), jnp.float32),
            pltpu.VMEM_SHARED((NP, DH), jnp.float32),
            [pltpu.SemaphoreType.DMA] * NBUF,
        ],
    )(g, src3, dst3)


def _deg_kernel(dst_hbm, out_hbm, dst_v, ones_v, zbuf_v, acc_sh):
    c = lax.axis_index("c")
    s = lax.axis_index("s")

    _zero_fill(zbuf_v, 64, 32)
    one16 = jnp.ones((16,), jnp.float32)

    def fill_ones(i, _):
        ones_v[i // 2, pl.ds((i % 2) * 16, 16)] = one16
        return 0

    lax.fori_loop(0, 256, fill_ones, 0)

    base = s * ROWS_PER_TILE
    for k in range(ROWS_PER_TILE // 64):
        pltpu.sync_copy(zbuf_v, acc_sh.at[pl.ds(base + k * 64, 64)])
    pltpu.sync_copy(dst_hbm.at[s], dst_v)
    plsc.subcore_barrier()

    half = CHUNKS // 2

    def body(j, _):
        pltpu.sync_copy(ones_v, acc_sh.at[dst_v.at[c * half + j]], add=True)
        return 0

    lax.fori_loop(0, half, body, 0)
    plsc.subcore_barrier()
    pltpu.sync_copy(acc_sh.at[pl.ds(base, ROWS_PER_TILE)],
                    out_hbm.at[c, pl.ds(base, ROWS_PER_TILE)])


def _sc_deg(dst3):
    mesh = plsc.VectorSubcoreMesh(core_axis_name="c", subcore_axis_name="s")
    return pl.kernel(
        _deg_kernel,
        mesh=mesh,
        compiler_params=pltpu.CompilerParams(use_tc_tiling_on_sc=False),
        out_type=jax.ShapeDtypeStruct((NC, NP, 32), jnp.float32),
        scratch_types=[
            pltpu.VMEM((CHUNKS, 128), jnp.int32),
            pltpu.VMEM((128, 32), jnp.float32),
            pltpu.VMEM((64, 32), jnp.float32),
            pltpu.VMEM_SHARED((NP, 32), jnp.float32),
        ],
    )(dst3)


NR = NP // 4


def _kron4(wsub, rows):
    t = jnp.concatenate([wsub] * 4, axis=0)
    t = jnp.concatenate([t] * 4, axis=1)
    ri = lax.broadcasted_iota(jnp.int32, t.shape, 0) // (rows // 4)
    ci = lax.broadcasted_iota(jnp.int32, t.shape, 1) // 32
    return jnp.where(ri == ci, t, 0.0)


def _dinv_packed(degp_ref):
    deg = degp_ref[0] + degp_ref[1] + 1.0
    return lax.rsqrt(deg)


def _g0_body(x_ref, w_ref, degp_ref, o_ref):
    dinv = _dinv_packed(degp_ref)
    x2 = x_ref[...]
    for h in range(2):
        bd = _kron4(w_ref[:, h * DH:(h + 1) * DH], 512)
        o_ref[h] = dinv * jnp.dot(x2, bd, preferred_element_type=jnp.float32)


def _edges_body(ei_ref, src_ref, dst_ref):
    src_ref[pl.ds(0, E)] = ei_ref[0, :]
    dst_ref[pl.ds(0, E)] = ei_ref[1, :]
    src_ref[pl.ds(E, EP - E)] = jnp.full((EP - E,), N_NODES, jnp.int32)
    dst_ref[pl.ds(E, EP - E)] = jnp.full((EP - E,), NP - 1, jnp.int32)


def _tc_edges(edge_index):
    return pl.pallas_call(
        _edges_body,
        out_shape=[
            jax.ShapeDtypeStruct((EP,), jnp.int32),
            jax.ShapeDtypeStruct((EP,), jnp.int32),
        ],
    )(edge_index)


def _tc_g0(x2, W0, degp):
    return pl.pallas_call(
        _g0_body,
        out_shape=jax.ShapeDtypeStruct((NC, NR, 128), jnp.float32),
    )(x2, W0, degp)


def _relu_halves(s_ref, g_ref, dinv, b_ref):
    rs = []
    for h in range(2):
        bh = jnp.concatenate([b_ref[:, h * DH:(h + 1) * DH]] * 4, axis=1)
        rs.append(jnp.maximum(dinv * (s_ref[h] + g_ref[h]) + bh, 0.0))
    return rs


def _mid_body(s_ref, g_ref, degp_ref, b_ref, w_ref, o_ref):
    dinv = _dinv_packed(degp_ref)
    r = _relu_halves(s_ref, g_ref, dinv, b_ref)
    for h in range(2):
        acc = jnp.zeros((NR, 128), jnp.float32)
        for i in range(2):
            bd = _kron4(w_ref[i * DH:(i + 1) * DH, h * DH:(h + 1) * DH], 128)
            acc += jnp.dot(r[i], bd, preferred_element_type=jnp.float32)
        o_ref[h] = dinv * acc


def _tc_mid(S, g, degp, b, W):
    return pl.pallas_call(
        _mid_body,
        out_shape=jax.ShapeDtypeStruct((NC, NR, 128), jnp.float32),
    )(S, g, degp, b, W)


def _final_body(s_ref, g_ref, degp_ref, b_ref, batchp_ref, wlin_ref, blin_ref,
                o_ref):
    dinv = _dinv_packed(degp_ref)
    r = _relu_halves(s_ref, g_ref, dinv, b_ref)
    gid = lax.broadcasted_iota(jnp.int32, (NUM_GRAPHS, NR), 0)
    sums = []
    cnts = jnp.zeros((NUM_GRAPHS, 1), jnp.float32)
    for k in range(4):
        oh = (gid == batchp_ref[k:k + 1, :]).astype(jnp.float32)
        sums.append([jnp.dot(oh, r[h][:, k * DH:(k + 1) * DH],
                             preferred_element_type=jnp.float32)
                     for h in range(2)])
        cnts += jnp.sum(oh, axis=1, keepdims=True)
    pooled = jnp.concatenate(
        [sums[0][0] + sums[1][0] + sums[2][0] + sums[3][0],
         sums[0][1] + sums[1][1] + sums[2][1] + sums[3][1]],
        axis=1) / jnp.maximum(cnts, 1.0)
    o_ref[...] = jnp.dot(pooled, wlin_ref[...],
                         preferred_element_type=jnp.float32) + blin_ref[...]


def _tc_final(S, g, degp, b, batchp, Wlin, blin):
    return pl.pallas_call(
        _final_body,
        out_shape=jax.ShapeDtypeStruct((NUM_GRAPHS, Wlin.shape[1]),
                                       jnp.float32),
    )(S, g, degp, b, batchp, Wlin, blin)


@jax.jit
def kernel(x, edge_index, batch, W0, b0, W1, b1, W2, b2, Wlin, blin):
    n = x.shape[0]
    xp = jnp.zeros((NP, x.shape[1]), x.dtype).at[:n].set(x)
    src, dst = _tc_edges(edge_index)
    src3 = src.reshape(NS, CHUNKS, 128)
    dst3 = dst.reshape(NS, CHUNKS, 128)
    bb = jnp.full((NP,), NUM_GRAPHS + 7, jnp.int32).at[:n].set(batch)
    batchp = bb.reshape(NR, 4).T

    def to_sc(a):
        return a.reshape(NC, NP, DH)

    def to_tc(a):
        return a.reshape(NC, NR, 128)

    degp = to_tc(_sc_deg(dst3))
    g0 = _tc_g0(xp.reshape(NR, 512), W0, degp)
    S0 = to_tc(_sc_scatter(to_sc(g0), src3, dst3))
    g1 = _tc_mid(S0, g0, degp, b0.reshape(1, D), W1)
    S1 = to_tc(_sc_scatter(to_sc(g1), src3, dst3))
    g2 = _tc_mid(S1, g1, degp, b1.reshape(1, D), W2)
    S2 = to_tc(_sc_scatter(to_sc(g2), src3, dst3))
    out = _tc_final(S2, g2, degp, b2.reshape(1, D), batchp,
                    Wlin, blin.reshape(1, -1))
    return out
